# baseline (speedup 1.0000x reference)
import jax
import jax.numpy as jnp
from jax.experimental import pallas as pl
from jax.experimental.pallas import tpu as pltpu

_B = 16
_L = 4096
_D = 1024
_CH = 512
_NBUF = 4
_T_MAX = _B * (_L // _CH)


def _body(lens_ref, rows_ref, chks_ref, t_ref, x_hbm, o_hbm):
    T = t_ref[0]

    def inner(idxs, x_blk, o_blk):
        (t,) = idxs
        r = rows_ref[t]
        chk = chks_ref[t]

        @pl.when(t == 0)
        def _init():
            o_blk[...] = jnp.zeros_like(o_blk)

        rel = lens_ref[r] - chk * _CH
        rowsi = jax.lax.broadcasted_iota(jnp.int32, (1, _CH, 1), 1)
        x = jnp.where(rowsi < rel, x_blk[...], 0.0)
        ps = jnp.sum(x, axis=1)
        o_blk[pl.ds(r, 1), :] += ps

        @pl.when(t == T - 1)
        def _fin():
            for i in range(_B):
                o_blk[pl.ds(i, 1), :] = (
                    o_blk[pl.ds(i, 1), :] / lens_ref[i].astype(jnp.float32)
                )

    pipe = pltpu.emit_pipeline(
        inner,
        grid=(T,),
        in_specs=[
            pl.BlockSpec(
                (1, _CH, _D),
                lambda t: (rows_ref[t], chks_ref[t], 0),
                pipeline_mode=pl.Buffered(buffer_count=_NBUF),
            )
        ],
        out_specs=[pl.BlockSpec((_B, _D), lambda t: (0, 0))],
        _explicit_indices=True,
    )
    pipe(x_hbm, o_hbm)


@jax.jit
def kernel(V, atoms_lens):
    E = V[0]
    lens = atoms_lens.astype(jnp.int32)
    nb = (lens + _CH - 1) // _CH
    prefix = jnp.cumsum(nb).astype(jnp.int32)
    T = jnp.maximum(prefix[-1], 1)
    t_arr = jnp.arange(_T_MAX, dtype=jnp.int32)
    row = jnp.minimum(
        jnp.searchsorted(prefix, t_arr, side="right").astype(jnp.int32), _B - 1
    )
    start = jnp.concatenate([jnp.zeros((1,), jnp.int32), prefix[:-1]])
    chk = jnp.clip(t_arr - start[row], 0, _L // _CH - 1)

    grid_spec = pltpu.PrefetchScalarGridSpec(
        num_scalar_prefetch=4,
        grid=(1,),
        in_specs=[pl.BlockSpec(memory_space=pl.ANY)],
        out_specs=pl.BlockSpec(memory_space=pl.ANY),
    )
    return pl.pallas_call(
        _body,
        grid_spec=grid_spec,
        out_shape=jax.ShapeDtypeStruct((_B, _D), jnp.float32),
    )(lens, row, chk, T.reshape(1), E)
